# baseline (speedup 1.0000x reference)
import functools

import jax
import jax.numpy as jnp
from jax.experimental import pallas as pl
from jax.experimental.pallas import tpu as pltpu
from jax.experimental.pallas import tpu_sc as plsc

N = 10000
NH = N // 2
IN_CH = 128
REP = 128
HEADS = 8
K = 6
SLOTS = 8
BR1 = 200
BR2 = 200
BR4 = 400

NEG = -1e30


def _leaky(e):
    return jnp.where(e > 0, e, 0.2 * e)


def _topk_body(off, x_ref, xT_ref, nbr_ref, mb_ref, s_ref):
    i0 = (pl.program_id(0) + off) * BR1
    sm = jnp.dot(x_ref[...], xT_ref[...],
                 preferred_element_type=jnp.float32)
    col = jax.lax.broadcasted_iota(jnp.int32, (BR1, N), 1)
    rows = i0 + jax.lax.broadcasted_iota(jnp.int32, (BR1, 1), 0)
    vals, sels = [], []
    for k in range(K):
        if k > 0:
            sm = jnp.where(col == sels[k - 1], -jnp.inf, sm)
        m = jnp.max(sm, axis=1, keepdims=True)
        cand = jnp.where(sm == m, col, N)
        sel = jnp.min(cand, axis=1, keepdims=True)
        vals.append(m)
        sels.append(sel)
    v6 = jnp.concatenate(vals, axis=1)
    i6 = jnp.concatenate(sels, axis=1)
    valid = (i6 > rows) & (v6 != 0.0)
    nbr_ref[...] = jnp.concatenate(
        [jnp.where(valid, i6, rows), rows, rows], axis=1)
    zeros = jnp.zeros((BR1, 1), jnp.float32)
    mb_ref[...] = jnp.concatenate(
        [jnp.where(valid, 0.0, NEG), zeros, zeros + NEG], axis=1)


def _topk_half(x, xT, half):
    off = half * (NH // BR1)
    return pl.pallas_call(
        functools.partial(_topk_body, off),
        grid=(NH // BR1,),
        in_specs=[
            pl.BlockSpec((BR1, IN_CH), lambda i, o=off: (i + o, 0)),
            pl.BlockSpec((IN_CH, N), lambda i: (0, 0)),
        ],
        out_specs=[
            pl.BlockSpec((BR1, SLOTS), lambda i: (i, 0)),
            pl.BlockSpec((BR1, SLOTS), lambda i: (i, 0)),
        ],
        out_shape=[
            jax.ShapeDtypeStruct((NH, SLOTS), jnp.int32),
            jax.ShapeDtypeStruct((NH, SLOTS), jnp.float32),
        ],
        scratch_shapes=[pltpu.VMEM((BR1, N), jnp.float32)],
        compiler_params=pltpu.CompilerParams(
            dimension_semantics=("parallel",)),
    )(x, xT)


_GW = 128


def _gather(table, idx_flat):
    m = idx_flat.shape[1]
    total = ((m + _GW - 1) // _GW) * _GW
    if total != m:
        idx_flat = jnp.pad(idx_flat, ((0, 0), (0, total - m)))
    mesh = plsc.VectorSubcoreMesh(core_axis_name="core",
                                  subcore_axis_name="subcore")

    @pl.kernel(
        out_type=jax.ShapeDtypeStruct((total, table.shape[1]), table.dtype),
        mesh=mesh)
    def kern(tab_hbm, i_hbm, o_hbm):
        def body(i_vmem, o_vmem):
            pltpu.sync_copy(tab_hbm.at[i_vmem.at[0]], o_vmem)

        pltpu.emit_pipeline(
            body,
            grid=(total // _GW,),
            in_specs=[pl.BlockSpec((1, _GW), index_map=lambda i: (0, i))],
            out_specs=[pl.BlockSpec((_GW, table.shape[1]),
                                    index_map=lambda i: (i, 0))],
            core_axis_name=("core", "subcore"),
            dimension_semantics=(pltpu.PARALLEL,),
        )(i_hbm, o_hbm)

    return kern(table, idx_flat)[:m]


def _layer1_body(x_ref, xg_ref, mb_ref, W1l_ref, W1r_ref, A1_ref, b1_ref,
                 a1_ref, W2l_ref, W2r_ref, yl_ref, yr_ref):
    xr = jnp.dot(x_ref[...], W1r_ref[...],
                 preferred_element_type=jnp.float32)
    alphas = []
    for s in range(SLOTS):
        xlg = jnp.dot(xg_ref[s], W1l_ref[...],
                      preferred_element_type=jnp.float32)
        e = _leaky(xlg + xr)
        al = jnp.dot(e, A1_ref[...],
                     preferred_element_type=jnp.float32)
        alphas.append(al + mb_ref[:, s:s + 1])
    m = alphas[0]
    for s in range(1, SLOTS):
        m = jnp.maximum(m, alphas[s])
    exs = [jnp.exp(al - m) for al in alphas]
    den = exs[0]
    for s in range(1, SLOTS):
        den = den + exs[s]
    acc = jnp.zeros((BR2, HEADS * REP), jnp.float32)
    for s in range(SLOTS):
        w = exs[s] / den
        wf = jnp.broadcast_to(w[:, :, None], (BR2, HEADS, REP))
        xlg = jnp.dot(xg_ref[s], W1l_ref[...],
                      preferred_element_type=jnp.float32)
        acc = acc + wf.reshape(BR2, HEADS * REP) * xlg
    h = acc + b1_ref[...]
    h = jnp.where(h > 0, h, a1_ref[0, 0] * h)
    yl_ref[...] = jnp.dot(h, W2l_ref[...],
                          preferred_element_type=jnp.float32)
    yr_ref[...] = jnp.dot(h, W2r_ref[...],
                          preferred_element_type=jnp.float32)


def _layer1_half(x, xg, mb, W1l, W1r, A1, b1, a1, W2l, W2r, half):
    off = half * (NH // BR2)
    D1 = HEADS * REP
    return pl.pallas_call(
        _layer1_body,
        grid=(NH // BR2,),
        in_specs=[
            pl.BlockSpec((BR2, IN_CH), lambda i, o=off: (i + o, 0)),
            pl.BlockSpec((SLOTS, BR2, IN_CH), lambda i: (0, i, 0)),
            pl.BlockSpec((BR2, SLOTS), lambda i: (i, 0)),
            pl.BlockSpec((IN_CH, D1), lambda i: (0, 0)),
            pl.BlockSpec((IN_CH, D1), lambda i: (0, 0)),
            pl.BlockSpec((D1, HEADS), lambda i: (0, 0)),
            pl.BlockSpec((1, D1), lambda i: (0, 0)),
            pl.BlockSpec((1, 1), lambda i: (0, 0)),
            pl.BlockSpec((D1, REP), lambda i: (0, 0)),
            pl.BlockSpec((D1, REP), lambda i: (0, 0)),
        ],
        out_specs=[
            pl.BlockSpec((BR2, REP), lambda i: (i, 0)),
            pl.BlockSpec((BR2, REP), lambda i: (i, 0)),
        ],
        out_shape=[
            jax.ShapeDtypeStruct((NH, REP), jnp.float32),
            jax.ShapeDtypeStruct((NH, REP), jnp.float32),
        ],
        compiler_params=pltpu.CompilerParams(
            dimension_semantics=("parallel",)),
    )(x, xg, mb, W1l, W1r, A1, b1, a1, W2l, W2r)


def _layer2_body(ylg_ref, yr_ref, mb_ref, A2_ref, b2_ref, a2_ref, o_ref):
    yr = yr_ref[...]
    ylg32 = [ylg_ref[s].astype(jnp.float32) for s in range(SLOTS)]
    alphas = []
    for s in range(SLOTS):
        e = _leaky(ylg32[s] + yr)
        al = jnp.dot(e, A2_ref[...],
                     preferred_element_type=jnp.float32)
        alphas.append(al[:, :1] + mb_ref[:, s:s + 1])
    m = alphas[0]
    for s in range(1, SLOTS):
        m = jnp.maximum(m, alphas[s])
    exs = [jnp.exp(al - m) for al in alphas]
    den = exs[0]
    for s in range(1, SLOTS):
        den = den + exs[s]
    acc = jnp.zeros((BR4, REP), jnp.float32)
    for s in range(SLOTS):
        acc = acc + (exs[s] / den) * ylg32[s]
    out = acc + b2_ref[...]
    o_ref[...] = jnp.where(out > 0, out, a2_ref[0, 0] * out)


def _layer2(ylg, yr, mb, A2, b2, a2):
    return pl.pallas_call(
        _layer2_body,
        grid=(N // BR4,),
        in_specs=[
            pl.BlockSpec((SLOTS, BR4, REP), lambda i: (0, i, 0)),
            pl.BlockSpec((BR4, REP), lambda i: (i, 0)),
            pl.BlockSpec((BR4, SLOTS), lambda i: (i, 0)),
            pl.BlockSpec((REP, SLOTS), lambda i: (0, 0)),
            pl.BlockSpec((1, REP), lambda i: (0, 0)),
            pl.BlockSpec((1, 1), lambda i: (0, 0)),
        ],
        out_specs=pl.BlockSpec((BR4, REP), lambda i: (i, 0)),
        out_shape=jax.ShapeDtypeStruct((N, REP), jnp.float32),
        compiler_params=pltpu.CompilerParams(
            dimension_semantics=("parallel",)),
    )(ylg, yr, mb, A2, b2, a2)


def kernel(x, W1l, W1r, att1, b1, a1, W2l, W2r, att2, b2, a2):
    xT = x.T
    A1 = (jnp.eye(HEADS, dtype=jnp.float32)[:, None, :]
          * att1[:, :, None]).reshape(HEADS * REP, HEADS)
    A2 = jnp.pad(att2.reshape(REP, 1), ((0, 0), (0, SLOTS - 1)))
    b1r = b1.reshape(1, HEADS * REP)
    a1r = jnp.reshape(a1, (1, 1))

    nbr_a, mb_a = _topk_half(x, xT, 0)
    idx_a = nbr_a.T.reshape(1, SLOTS * NH)
    xg_a = _gather(x, idx_a).reshape(SLOTS, NH, IN_CH)
    nbr_b, mb_b = _topk_half(x, xT, 1)
    idx_b = nbr_b.T.reshape(1, SLOTS * NH)
    xg_b = _gather(x, idx_b).reshape(SLOTS, NH, IN_CH)
    yl_a, yr_a = _layer1_half(x, xg_a, mb_a, W1l, W1r, A1, b1r, a1r,
                              W2l, W2r, 0)
    yl_b, yr_b = _layer1_half(x, xg_b, mb_b, W1l, W1r, A1, b1r, a1r,
                              W2l, W2r, 1)
    yl = jnp.concatenate([yl_a, yl_b], axis=0)
    yr = jnp.concatenate([yr_a, yr_b], axis=0)
    mb = jnp.concatenate([mb_a, mb_b], axis=0)
    idx = jnp.concatenate(
        [nbr_a, nbr_b], axis=0).T.reshape(1, SLOTS * N)
    ylg = _gather(yl, idx).reshape(SLOTS, N, REP)
    out = _layer2(ylg, yr, mb, A2,
                  b2.reshape(1, REP),
                  jnp.reshape(a2, (1, 1)))
    return out


# speedup vs baseline: 1.0360x; 1.0360x over previous
import functools

import jax
import jax.numpy as jnp
from jax.experimental import pallas as pl
from jax.experimental.pallas import tpu as pltpu
from jax.experimental.pallas import tpu_sc as plsc

N = 10000
NHA = 5200
NHB = 4800
IN_CH = 128
REP = 128
HEADS = 8
K = 6
SLOTS = 8
BR1 = 200
BR2 = 200
BR4 = 400

NEG = -1e30


def _leaky(e):
    return jnp.where(e > 0, e, 0.2 * e)


def _topk_body(off, x_ref, xT_ref, nbr_ref, mb_ref, s_ref):
    i0 = (pl.program_id(0) + off) * BR1
    sm = jnp.dot(x_ref[...], xT_ref[...],
                 preferred_element_type=jnp.float32)
    col = jax.lax.broadcasted_iota(jnp.int32, (BR1, N), 1)
    rows = i0 + jax.lax.broadcasted_iota(jnp.int32, (BR1, 1), 0)
    vals, sels = [], []
    for k in range(K):
        if k > 0:
            sm = jnp.where(col == sels[k - 1], -jnp.inf, sm)
        m = jnp.max(sm, axis=1, keepdims=True)
        cand = jnp.where(sm == m, col, N)
        sel = jnp.min(cand, axis=1, keepdims=True)
        vals.append(m)
        sels.append(sel)
    v6 = jnp.concatenate(vals, axis=1)
    i6 = jnp.concatenate(sels, axis=1)
    valid = (i6 > rows) & (v6 != 0.0)
    nbr_ref[...] = jnp.concatenate(
        [jnp.where(valid, i6, rows), rows, rows], axis=1)
    zeros = jnp.zeros((BR1, 1), jnp.float32)
    mb_ref[...] = jnp.concatenate(
        [jnp.where(valid, 0.0, NEG), zeros, zeros + NEG], axis=1)


def _topk_half(x, xT, off, nrows):
    return pl.pallas_call(
        functools.partial(_topk_body, off),
        grid=(nrows // BR1,),
        in_specs=[
            pl.BlockSpec((BR1, IN_CH), lambda i, o=off: (i + o, 0)),
            pl.BlockSpec((IN_CH, N), lambda i: (0, 0)),
        ],
        out_specs=[
            pl.BlockSpec((BR1, SLOTS), lambda i: (i, 0)),
            pl.BlockSpec((BR1, SLOTS), lambda i: (i, 0)),
        ],
        out_shape=[
            jax.ShapeDtypeStruct((nrows, SLOTS), jnp.int32),
            jax.ShapeDtypeStruct((nrows, SLOTS), jnp.float32),
        ],
        scratch_shapes=[pltpu.VMEM((BR1, N), jnp.float32)],
        compiler_params=pltpu.CompilerParams(
            dimension_semantics=("parallel",)),
    )(x, xT)


_GW = 128


def _gather(table, idx_flat):
    m = idx_flat.shape[1]
    total = ((m + _GW - 1) // _GW) * _GW
    if total != m:
        idx_flat = jnp.pad(idx_flat, ((0, 0), (0, total - m)))
    mesh = plsc.VectorSubcoreMesh(core_axis_name="core",
                                  subcore_axis_name="subcore")

    @pl.kernel(
        out_type=jax.ShapeDtypeStruct((total, table.shape[1]), table.dtype),
        mesh=mesh)
    def kern(tab_hbm, i_hbm, o_hbm):
        def body(i_vmem, o_vmem):
            pltpu.sync_copy(tab_hbm.at[i_vmem.at[0]], o_vmem)

        pltpu.emit_pipeline(
            body,
            grid=(total // _GW,),
            in_specs=[pl.BlockSpec((1, _GW), index_map=lambda i: (0, i))],
            out_specs=[pl.BlockSpec((_GW, table.shape[1]),
                                    index_map=lambda i: (i, 0))],
            core_axis_name=("core", "subcore"),
            dimension_semantics=(pltpu.PARALLEL,),
        )(i_hbm, o_hbm)

    return kern(table, idx_flat)[:m]


def _layer1_body(x_ref, xg_ref, mb_ref, W1l_ref, W1r_ref, A1_ref, b1_ref,
                 a1_ref, W2l_ref, W2r_ref, yl_ref, yr_ref):
    xr = jnp.dot(x_ref[...], W1r_ref[...],
                 preferred_element_type=jnp.float32)
    alphas = []
    for s in range(SLOTS):
        xlg = jnp.dot(xg_ref[s], W1l_ref[...],
                      preferred_element_type=jnp.float32)
        e = _leaky(xlg + xr)
        al = jnp.dot(e, A1_ref[...],
                     preferred_element_type=jnp.float32)
        alphas.append(al + mb_ref[:, s:s + 1])
    m = alphas[0]
    for s in range(1, SLOTS):
        m = jnp.maximum(m, alphas[s])
    exs = [jnp.exp(al - m) for al in alphas]
    den = exs[0]
    for s in range(1, SLOTS):
        den = den + exs[s]
    acc = jnp.zeros((BR2, HEADS * REP), jnp.float32)
    for s in range(SLOTS):
        w = exs[s] / den
        wf = jnp.broadcast_to(w[:, :, None], (BR2, HEADS, REP))
        xlg = jnp.dot(xg_ref[s], W1l_ref[...],
                      preferred_element_type=jnp.float32)
        acc = acc + wf.reshape(BR2, HEADS * REP) * xlg
    h = acc + b1_ref[...]
    h = jnp.where(h > 0, h, a1_ref[0, 0] * h)
    yl_ref[...] = jnp.dot(h, W2l_ref[...],
                          preferred_element_type=jnp.float32)
    yr_ref[...] = jnp.dot(h, W2r_ref[...],
                          preferred_element_type=jnp.float32)


def _layer1_half(x, xg, mb, W1l, W1r, A1, b1, a1, W2l, W2r, off, nrows):
    D1 = HEADS * REP
    return pl.pallas_call(
        _layer1_body,
        grid=(nrows // BR2,),
        in_specs=[
            pl.BlockSpec((BR2, IN_CH), lambda i, o=off: (i + o, 0)),
            pl.BlockSpec((SLOTS, BR2, IN_CH), lambda i: (0, i, 0)),
            pl.BlockSpec((BR2, SLOTS), lambda i: (i, 0)),
            pl.BlockSpec((IN_CH, D1), lambda i: (0, 0)),
            pl.BlockSpec((IN_CH, D1), lambda i: (0, 0)),
            pl.BlockSpec((D1, HEADS), lambda i: (0, 0)),
            pl.BlockSpec((1, D1), lambda i: (0, 0)),
            pl.BlockSpec((1, 1), lambda i: (0, 0)),
            pl.BlockSpec((D1, REP), lambda i: (0, 0)),
            pl.BlockSpec((D1, REP), lambda i: (0, 0)),
        ],
        out_specs=[
            pl.BlockSpec((BR2, REP), lambda i: (i, 0)),
            pl.BlockSpec((BR2, REP), lambda i: (i, 0)),
        ],
        out_shape=[
            jax.ShapeDtypeStruct((nrows, REP), jnp.float32),
            jax.ShapeDtypeStruct((nrows, REP), jnp.float32),
        ],
        compiler_params=pltpu.CompilerParams(
            dimension_semantics=("parallel",)),
    )(x, xg, mb, W1l, W1r, A1, b1, a1, W2l, W2r)


def _layer2_body(ylg_ref, yr_ref, mb_ref, A2_ref, b2_ref, a2_ref, o_ref):
    yr = yr_ref[...]
    ylg32 = [ylg_ref[s].astype(jnp.float32) for s in range(SLOTS)]
    alphas = []
    for s in range(SLOTS):
        e = _leaky(ylg32[s] + yr)
        al = jnp.dot(e, A2_ref[...],
                     preferred_element_type=jnp.float32)
        alphas.append(al[:, :1] + mb_ref[:, s:s + 1])
    m = alphas[0]
    for s in range(1, SLOTS):
        m = jnp.maximum(m, alphas[s])
    exs = [jnp.exp(al - m) for al in alphas]
    den = exs[0]
    for s in range(1, SLOTS):
        den = den + exs[s]
    acc = jnp.zeros((BR4, REP), jnp.float32)
    for s in range(SLOTS):
        acc = acc + (exs[s] / den) * ylg32[s]
    out = acc + b2_ref[...]
    o_ref[...] = jnp.where(out > 0, out, a2_ref[0, 0] * out)


def _layer2(ylg, yr, mb, A2, b2, a2):
    return pl.pallas_call(
        _layer2_body,
        grid=(N // BR4,),
        in_specs=[
            pl.BlockSpec((SLOTS, BR4, REP), lambda i: (0, i, 0)),
            pl.BlockSpec((BR4, REP), lambda i: (i, 0)),
            pl.BlockSpec((BR4, SLOTS), lambda i: (i, 0)),
            pl.BlockSpec((REP, SLOTS), lambda i: (0, 0)),
            pl.BlockSpec((1, REP), lambda i: (0, 0)),
            pl.BlockSpec((1, 1), lambda i: (0, 0)),
        ],
        out_specs=pl.BlockSpec((BR4, REP), lambda i: (i, 0)),
        out_shape=jax.ShapeDtypeStruct((N, REP), jnp.float32),
        compiler_params=pltpu.CompilerParams(
            dimension_semantics=("parallel",)),
    )(ylg, yr, mb, A2, b2, a2)


def kernel(x, W1l, W1r, att1, b1, a1, W2l, W2r, att2, b2, a2):
    xT = x.T
    A1 = (jnp.eye(HEADS, dtype=jnp.float32)[:, None, :]
          * att1[:, :, None]).reshape(HEADS * REP, HEADS)
    A2 = jnp.pad(att2.reshape(REP, 1), ((0, 0), (0, SLOTS - 1)))
    b1r = b1.reshape(1, HEADS * REP)
    a1r = jnp.reshape(a1, (1, 1))

    nbr_a, mb_a = _topk_half(x, xT, 0, NHA)
    idx_a = nbr_a.T.reshape(1, SLOTS * NHA)
    xg_a = _gather(x, idx_a).reshape(SLOTS, NHA, IN_CH)
    nbr_b, mb_b = _topk_half(x, xT, NHA // BR1, NHB)
    idx_b = nbr_b.T.reshape(1, SLOTS * NHB)
    xg_b = _gather(x, idx_b).reshape(SLOTS, NHB, IN_CH)
    yl_a, yr_a = _layer1_half(x, xg_a, mb_a, W1l, W1r, A1, b1r, a1r,
                              W2l, W2r, 0, NHA)
    yl_b, yr_b = _layer1_half(x, xg_b, mb_b, W1l, W1r, A1, b1r, a1r,
                              W2l, W2r, NHA // BR2, NHB)
    yl = jnp.concatenate([yl_a, yl_b], axis=0)
    yr = jnp.concatenate([yr_a, yr_b], axis=0)
    mb = jnp.concatenate([mb_a, mb_b], axis=0)
    idx = jnp.concatenate(
        [nbr_a, nbr_b], axis=0).T.reshape(1, SLOTS * N)
    ylg = _gather(yl, idx).reshape(SLOTS, N, REP)
    out = _layer2(ylg, yr, mb, A2,
                  b2.reshape(1, REP),
                  jnp.reshape(a2, (1, 1)))
    return out
